# Initial kernel scaffold; baseline (speedup 1.0000x reference)
import jax
import jax.numpy as jnp
from jax.experimental import pallas as pl


def kernel(position_ids, table):
    raise NotImplementedError("write your pallas kernel here")


# speedup vs baseline: 1.6127x; 1.6127x over previous
import functools

import jax
import jax.numpy as jnp
from jax import lax
from jax.experimental import pallas as pl
from jax.experimental.pallas import tpu as pltpu
from jax.experimental.pallas import tpu_sc as plsc

_NC = 2
_NS = 16
_NW = _NC * _NS

_CH = 16
_NBUF = 2


@functools.partial(jax.jit, static_argnames=("b", "d"))
def _sc_gather(table, ids_flat, b, d):
    b_per_w = b // _NW
    n_ch = b_per_w // _CH
    mesh = plsc.VectorSubcoreMesh(core_axis_name="c", subcore_axis_name="s")

    @functools.partial(
        pl.kernel,
        out_type=jax.ShapeDtypeStruct((b, d), jnp.float32),
        mesh=mesh,
        scratch_types=[
            pltpu.VMEM((b_per_w,), jnp.int32),
            pltpu.VMEM((_CH, d), jnp.float32),
            pltpu.VMEM((_CH, d), jnp.float32),
            pltpu.SemaphoreType.DMA,
            pltpu.SemaphoreType.DMA,
        ],
    )
    def k(table_hbm, idx_hbm, out_hbm, idx_v, buf0, buf1, sem0, sem1):
        wid = lax.axis_index("s") * _NC + lax.axis_index("c")
        base = wid * b_per_w
        pltpu.sync_copy(idx_hbm.at[pl.ds(base, b_per_w)], idx_v)

        bufs = (buf0, buf1)
        sems = (sem0, sem1)

        def gather_start(c, slot):
            off = pl.multiple_of(c * _CH, 8)
            pltpu.async_copy(
                table_hbm.at[idx_v.at[pl.ds(off, _CH)]], bufs[slot], sems[slot]
            )

        def gather_wait(c, slot):
            off = pl.multiple_of(c * _CH, 8)
            pltpu.make_async_copy(
                table_hbm.at[idx_v.at[pl.ds(off, _CH)]], bufs[slot], sems[slot]
            ).wait()

        def store(c, slot):
            off = pl.multiple_of(base + c * _CH, 8)
            pltpu.sync_copy(bufs[slot], out_hbm.at[pl.ds(off, _CH)])

        for s in range(_NBUF):
            gather_start(s, s)

        def body(g, carry):
            for s in range(_NBUF):
                c = g * _NBUF + s
                gather_wait(c, s)
                store(c, s)

                @pl.when(g < n_ch // _NBUF - 1)
                def _():
                    gather_start(c + _NBUF, s)

            return carry

        lax.fori_loop(0, n_ch // _NBUF, body, 0)

    return k(table, ids_flat)


def kernel(position_ids, table):
    bsz, seq = position_ids.shape
    _, d = table.shape
    ids_flat = position_ids.reshape(-1).astype(jnp.int32)
    out = _sc_gather(table, ids_flat, bsz * seq, d)
    return out.reshape(bsz, seq, d)
